# speedup vs baseline: 11.0066x; 11.0066x over previous
import functools

import jax
import jax.numpy as jnp
from jax import lax
from jax.experimental import pallas as pl
from jax.experimental.pallas import tpu as pltpu
from jax.experimental.pallas import tpu_sc as plsc

N_NODES = 10000
N_FEAT = 128
N_CLASS = 64
N_EDGES = 320000

NPAD = 10240
NW = 32
EPW = N_EDGES // NW
K = 80
NCHUNK = EPW // K
RPT = NPAD // 16

_MESH = plsc.VectorSubcoreMesh(core_axis_name="c", subcore_axis_name="s")


@functools.partial(
    pl.kernel,
    mesh=_MESH,
    out_type=jax.ShapeDtypeStruct((2, NPAD), jnp.float32),
    scratch_types=[
        pltpu.VMEM((K,), jnp.int32),
        pltpu.VMEM((K,), jnp.float32),
        pltpu.VMEM_SHARED((NPAD,), jnp.float32),
    ],
)
def _sc_degree(dst_hbm, ones_hbm, zeros_hbm, out_hbm, dst_v, ones_v, acc_sh):
    c = lax.axis_index("c")
    s = lax.axis_index("s")
    r0 = s * RPT
    pltpu.sync_copy(zeros_hbm.at[pl.ds(r0, RPT)], acc_sh.at[pl.ds(r0, RPT)])
    pltpu.sync_copy(ones_hbm.at[pl.ds(0, K)], ones_v)
    plsc.subcore_barrier()
    ebase = (s * 2 + c) * EPW

    def body(i, carry):
        base = ebase + i * K
        pltpu.sync_copy(dst_hbm.at[pl.ds(base, K)], dst_v)
        pltpu.sync_copy(ones_v, acc_sh.at[dst_v], add=True)
        return carry

    lax.fori_loop(0, NCHUNK, body, 0)
    plsc.subcore_barrier()
    pltpu.sync_copy(acc_sh.at[pl.ds(r0, RPT)], out_hbm.at[c, pl.ds(r0, RPT)])


@functools.partial(
    pl.kernel,
    mesh=_MESH,
    out_type=jax.ShapeDtypeStruct((2, NPAD, N_FEAT), jnp.float32),
    scratch_types=[
        pltpu.VMEM((K,), jnp.int32),
        pltpu.VMEM((K,), jnp.int32),
        pltpu.VMEM((K, N_FEAT), jnp.float32),
        pltpu.VMEM_SHARED((NPAD, N_FEAT), jnp.float32),
        pltpu.SemaphoreType.DMA,
    ],
)
def _sc_scatter(g_hbm, src_hbm, dst_hbm, zeros_hbm, out_hbm,
                idx_v, dst_v, rows_v, acc_sh, sem):
    c = lax.axis_index("c")
    s = lax.axis_index("s")
    r0 = s * RPT
    pltpu.sync_copy(zeros_hbm.at[pl.ds(r0, RPT)], acc_sh.at[pl.ds(r0, RPT)])
    plsc.subcore_barrier()
    ebase = (s * 2 + c) * EPW

    def body(i, carry):
        base = ebase + i * K
        pltpu.sync_copy(src_hbm.at[pl.ds(base, K)], idx_v)
        pltpu.async_copy(g_hbm.at[idx_v], rows_v, sem).wait()
        pltpu.sync_copy(dst_hbm.at[pl.ds(base, K)], dst_v)
        pltpu.sync_copy(rows_v, acc_sh.at[dst_v], add=True)
        return carry

    lax.fori_loop(0, NCHUNK, body, 0)
    plsc.subcore_barrier()
    pltpu.sync_copy(acc_sh.at[pl.ds(r0, RPT)], out_hbm.at[c, pl.ds(r0, RPT)])


_R = 1024
_GRID = NPAD // _R


def _m0_body(x_ref, w_ref, dinv_ref, g_ref):
    g_ref[...] = dinv_ref[...] * jnp.dot(
        x_ref[...], w_ref[...], preferred_element_type=jnp.float32)


def _mid_body(p_ref, g_ref, dinv_ref, b_ref, w_ref, h_ref, gn_ref):
    dinv = dinv_ref[...]
    h = jnp.maximum(dinv * (p_ref[0] + p_ref[1] + g_ref[...]) + b_ref[...], 0.0)
    h_ref[...] = h
    gn_ref[...] = dinv * jnp.dot(h, w_ref[...], preferred_element_type=jnp.float32)


def _fin_body(p_ref, g_ref, dinv_ref, b_ref, h1_ref, h2_ref, q_ref,
              wout_ref, bout_ref, out_ref, alpha_ref):
    dinv = dinv_ref[...]
    h3 = jnp.maximum(dinv * (p_ref[0] + p_ref[1] + g_ref[...]) + b_ref[...], 0.0)
    h1 = h1_ref[...]
    h2 = h2_ref[...]
    q = q_ref[...]
    s1 = jnp.sum(h1 * q, axis=1, keepdims=True)
    s2 = jnp.sum(h2 * q, axis=1, keepdims=True)
    s3 = jnp.sum(h3 * q, axis=1, keepdims=True)
    m = jnp.maximum(jnp.maximum(s1, s2), s3)
    e1 = jnp.exp(s1 - m)
    e2 = jnp.exp(s2 - m)
    e3 = jnp.exp(s3 - m)
    den = e1 + e2 + e3
    a1 = e1 / den
    a2 = e2 / den
    a3 = e3 / den
    hsum = a1 * h1 + a2 * h2 + a3 * h3
    out_ref[...] = jnp.dot(
        hsum, wout_ref[...], preferred_element_type=jnp.float32) + bout_ref[...]
    cols = lax.broadcasted_iota(jnp.int32, alpha_ref.shape, 1)
    alpha_ref[...] = jnp.where(
        cols == 0, a1, jnp.where(cols == 1, a2, jnp.where(cols == 2, a3, 0.0)))


def _row_spec(width):
    return pl.BlockSpec((_R, width), lambda i: (i, 0))


def _full_spec(shape):
    nd = len(shape)
    return pl.BlockSpec(shape, lambda i: (0,) * nd)


_P_SPEC = pl.BlockSpec((2, _R, N_FEAT), lambda i: (0, i, 0))

_m0 = pl.pallas_call(
    _m0_body,
    grid=(_GRID,),
    in_specs=[_row_spec(N_FEAT), _full_spec((N_FEAT, N_FEAT)), _row_spec(1)],
    out_specs=_row_spec(N_FEAT),
    out_shape=jax.ShapeDtypeStruct((NPAD, N_FEAT), jnp.float32),
)

_mid = pl.pallas_call(
    _mid_body,
    grid=(_GRID,),
    in_specs=[_P_SPEC, _row_spec(N_FEAT), _row_spec(1),
              _full_spec((1, N_FEAT)), _full_spec((N_FEAT, N_FEAT))],
    out_specs=[_row_spec(N_FEAT), _row_spec(N_FEAT)],
    out_shape=[jax.ShapeDtypeStruct((NPAD, N_FEAT), jnp.float32),
               jax.ShapeDtypeStruct((NPAD, N_FEAT), jnp.float32)],
)

_fin = pl.pallas_call(
    _fin_body,
    grid=(_GRID,),
    in_specs=[_P_SPEC, _row_spec(N_FEAT), _row_spec(1),
              _full_spec((1, N_FEAT)), _row_spec(N_FEAT), _row_spec(N_FEAT),
              _full_spec((1, N_FEAT)), _full_spec((N_FEAT, N_CLASS)),
              _full_spec((1, N_CLASS))],
    out_specs=[_row_spec(N_CLASS), _row_spec(N_FEAT)],
    out_shape=[jax.ShapeDtypeStruct((NPAD, N_CLASS), jnp.float32),
               jax.ShapeDtypeStruct((NPAD, N_FEAT), jnp.float32)],
)


def kernel(x, edge_index, W0, b0, W1, b1, W2, b2, q, Wout, bout):
    src = edge_index[0]
    dst = edge_index[1]
    xp = jnp.pad(x, ((0, NPAD - N_NODES), (0, 0)))
    zerosF = jnp.zeros((NPAD, N_FEAT), jnp.float32)
    zerosN = jnp.zeros((NPAD,), jnp.float32)
    onesK = jnp.ones((K,), jnp.float32)

    degp = _sc_degree(dst, onesK, zerosN)
    dinv = lax.rsqrt(degp[0] + degp[1] + 1.0).reshape(NPAD, 1)

    g0 = _m0(xp, W0, dinv)
    p0 = _sc_scatter(g0, src, dst, zerosF)
    h1, g1 = _mid(p0, g0, dinv, b0.reshape(1, -1), W1)
    p1 = _sc_scatter(g1, src, dst, zerosF)
    h2, g2 = _mid(p1, g1, dinv, b1.reshape(1, -1), W2)
    p2 = _sc_scatter(g2, src, dst, zerosF)
    out_pad, alpha_pad = _fin(p2, g2, dinv, b2.reshape(1, -1), h1, h2,
                              q.reshape(1, -1), Wout, bout.reshape(1, -1))
    return out_pad[:N_NODES], alpha_pad[:N_NODES, :3]


# speedup vs baseline: 16.8858x; 1.5342x over previous
import functools

import jax
import jax.numpy as jnp
from jax import lax
from jax.experimental import pallas as pl
from jax.experimental.pallas import tpu as pltpu
from jax.experimental.pallas import tpu_sc as plsc

N_NODES = 10000
N_FEAT = 128
N_CLASS = 64
N_EDGES = 320000

NPAD = 10240
NW = 32
EPW = N_EDGES // NW
K = 80
NCHUNK = EPW // K
RPT = NPAD // 16

_MESH = plsc.VectorSubcoreMesh(core_axis_name="c", subcore_axis_name="s")


@functools.partial(
    pl.kernel,
    mesh=_MESH,
    out_type=jax.ShapeDtypeStruct((2, NPAD), jnp.float32),
    scratch_types=[
        pltpu.VMEM((K,), jnp.int32),
        pltpu.VMEM((K,), jnp.float32),
        pltpu.VMEM_SHARED((NPAD,), jnp.float32),
    ],
)
def _sc_degree(dst_hbm, ones_hbm, zeros_hbm, out_hbm, dst_v, ones_v, acc_sh):
    c = lax.axis_index("c")
    s = lax.axis_index("s")
    r0 = s * RPT
    pltpu.sync_copy(zeros_hbm.at[pl.ds(r0, RPT)], acc_sh.at[pl.ds(r0, RPT)])
    pltpu.sync_copy(ones_hbm.at[pl.ds(0, K)], ones_v)
    plsc.subcore_barrier()
    ebase = (s * 2 + c) * EPW

    def body(i, carry):
        base = ebase + i * K
        pltpu.sync_copy(dst_hbm.at[pl.ds(base, K)], dst_v)
        pltpu.sync_copy(ones_v, acc_sh.at[dst_v], add=True)
        return carry

    lax.fori_loop(0, NCHUNK, body, 0)
    plsc.subcore_barrier()
    pltpu.sync_copy(acc_sh.at[pl.ds(r0, RPT)], out_hbm.at[c, pl.ds(r0, RPT)])


NB = 4
NGRP = -(-NCHUNK // NB)


@functools.partial(
    pl.kernel,
    mesh=_MESH,
    out_type=jax.ShapeDtypeStruct((2, NPAD, N_FEAT), jnp.float32),
    scratch_types=[
        pltpu.VMEM((NB * K,), jnp.int32),
        pltpu.VMEM((K,), jnp.int32),
    ] + [pltpu.VMEM((K, N_FEAT), jnp.float32)] * NB + [
        pltpu.VMEM_SHARED((NPAD, N_FEAT), jnp.float32),
    ] + [pltpu.SemaphoreType.DMA] * NB,
)
def _sc_scatter(g_hbm, src_hbm, dst_hbm, zeros_hbm, out_hbm,
                srcs_v, dst_v, *rest):
    rows = rest[:NB]
    acc_sh = rest[NB]
    sems = rest[NB + 1:]
    c = lax.axis_index("c")
    s = lax.axis_index("s")
    r0 = s * RPT
    pltpu.sync_copy(zeros_hbm.at[pl.ds(r0, RPT)], acc_sh.at[pl.ds(r0, RPT)])
    plsc.subcore_barrier()
    ebase = (s * 2 + c) * EPW

    for b in range(NB):
        pltpu.sync_copy(src_hbm.at[pl.ds(ebase + b * K, K)],
                        srcs_v.at[pl.ds(b * K, K)])
        pltpu.async_copy(g_hbm.at[srcs_v.at[pl.ds(b * K, K)]], rows[b], sems[b])

    def group(jg, carry):
        for b in range(NB):
            i = jg * NB + b

            @pl.when(i < NCHUNK)
            def _():
                pltpu.make_async_copy(
                    g_hbm.at[srcs_v.at[pl.ds(b * K, K)]], rows[b],
                    sems[b]).wait()
                pltpu.sync_copy(dst_hbm.at[pl.ds(ebase + i * K, K)], dst_v)
                pltpu.sync_copy(rows[b], acc_sh.at[dst_v], add=True)
                nxt = i + NB

                @pl.when(nxt < NCHUNK)
                def _():
                    pltpu.sync_copy(src_hbm.at[pl.ds(ebase + nxt * K, K)],
                                    srcs_v.at[pl.ds(b * K, K)])
                    pltpu.async_copy(
                        g_hbm.at[srcs_v.at[pl.ds(b * K, K)]], rows[b], sems[b])
        return carry

    lax.fori_loop(0, NGRP, group, 0)
    plsc.subcore_barrier()
    pltpu.sync_copy(acc_sh.at[pl.ds(r0, RPT)], out_hbm.at[c, pl.ds(r0, RPT)])


_R = 1024
_GRID = NPAD // _R


def _m0_body(x_ref, w_ref, dinv_ref, g_ref):
    g_ref[...] = dinv_ref[...] * jnp.dot(
        x_ref[...], w_ref[...], preferred_element_type=jnp.float32)


def _mid_body(p_ref, g_ref, dinv_ref, b_ref, w_ref, h_ref, gn_ref):
    dinv = dinv_ref[...]
    h = jnp.maximum(dinv * (p_ref[0] + p_ref[1] + g_ref[...]) + b_ref[...], 0.0)
    h_ref[...] = h
    gn_ref[...] = dinv * jnp.dot(h, w_ref[...], preferred_element_type=jnp.float32)


def _fin_body(p_ref, g_ref, dinv_ref, b_ref, h1_ref, h2_ref, q_ref,
              wout_ref, bout_ref, out_ref, alpha_ref):
    dinv = dinv_ref[...]
    h3 = jnp.maximum(dinv * (p_ref[0] + p_ref[1] + g_ref[...]) + b_ref[...], 0.0)
    h1 = h1_ref[...]
    h2 = h2_ref[...]
    q = q_ref[...]
    s1 = jnp.sum(h1 * q, axis=1, keepdims=True)
    s2 = jnp.sum(h2 * q, axis=1, keepdims=True)
    s3 = jnp.sum(h3 * q, axis=1, keepdims=True)
    m = jnp.maximum(jnp.maximum(s1, s2), s3)
    e1 = jnp.exp(s1 - m)
    e2 = jnp.exp(s2 - m)
    e3 = jnp.exp(s3 - m)
    den = e1 + e2 + e3
    a1 = e1 / den
    a2 = e2 / den
    a3 = e3 / den
    hsum = a1 * h1 + a2 * h2 + a3 * h3
    out_ref[...] = jnp.dot(
        hsum, wout_ref[...], preferred_element_type=jnp.float32) + bout_ref[...]
    cols = lax.broadcasted_iota(jnp.int32, alpha_ref.shape, 1)
    alpha_ref[...] = jnp.where(
        cols == 0, a1, jnp.where(cols == 1, a2, jnp.where(cols == 2, a3, 0.0)))


def _row_spec(width):
    return pl.BlockSpec((_R, width), lambda i: (i, 0))


def _full_spec(shape):
    nd = len(shape)
    return pl.BlockSpec(shape, lambda i: (0,) * nd)


_P_SPEC = pl.BlockSpec((2, _R, N_FEAT), lambda i: (0, i, 0))

_m0 = pl.pallas_call(
    _m0_body,
    grid=(_GRID,),
    in_specs=[_row_spec(N_FEAT), _full_spec((N_FEAT, N_FEAT)), _row_spec(1)],
    out_specs=_row_spec(N_FEAT),
    out_shape=jax.ShapeDtypeStruct((NPAD, N_FEAT), jnp.float32),
)

_mid = pl.pallas_call(
    _mid_body,
    grid=(_GRID,),
    in_specs=[_P_SPEC, _row_spec(N_FEAT), _row_spec(1),
              _full_spec((1, N_FEAT)), _full_spec((N_FEAT, N_FEAT))],
    out_specs=[_row_spec(N_FEAT), _row_spec(N_FEAT)],
    out_shape=[jax.ShapeDtypeStruct((NPAD, N_FEAT), jnp.float32),
               jax.ShapeDtypeStruct((NPAD, N_FEAT), jnp.float32)],
)

_fin = pl.pallas_call(
    _fin_body,
    grid=(_GRID,),
    in_specs=[_P_SPEC, _row_spec(N_FEAT), _row_spec(1),
              _full_spec((1, N_FEAT)), _row_spec(N_FEAT), _row_spec(N_FEAT),
              _full_spec((1, N_FEAT)), _full_spec((N_FEAT, N_CLASS)),
              _full_spec((1, N_CLASS))],
    out_specs=[_row_spec(N_CLASS), _row_spec(N_FEAT)],
    out_shape=[jax.ShapeDtypeStruct((NPAD, N_CLASS), jnp.float32),
               jax.ShapeDtypeStruct((NPAD, N_FEAT), jnp.float32)],
)


def kernel(x, edge_index, W0, b0, W1, b1, W2, b2, q, Wout, bout):
    src = edge_index[0]
    dst = edge_index[1]
    xp = jnp.pad(x, ((0, NPAD - N_NODES), (0, 0)))
    zerosF = jnp.zeros((NPAD, N_FEAT), jnp.float32)
    zerosN = jnp.zeros((NPAD,), jnp.float32)
    onesK = jnp.ones((K,), jnp.float32)

    degp = _sc_degree(dst, onesK, zerosN)
    dinv = lax.rsqrt(degp[0] + degp[1] + 1.0).reshape(NPAD, 1)

    g0 = _m0(xp, W0, dinv)
    p0 = _sc_scatter(g0, src, dst, zerosF)
    h1, g1 = _mid(p0, g0, dinv, b0.reshape(1, -1), W1)
    p1 = _sc_scatter(g1, src, dst, zerosF)
    h2, g2 = _mid(p1, g1, dinv, b1.reshape(1, -1), W2)
    p2 = _sc_scatter(g2, src, dst, zerosF)
    out_pad, alpha_pad = _fin(p2, g2, dinv, b2.reshape(1, -1), h1, h2,
                              q.reshape(1, -1), Wout, bout.reshape(1, -1))
    return out_pad[:N_NODES], alpha_pad[:N_NODES, :3]


# speedup vs baseline: 23.4916x; 1.3912x over previous
import functools

import jax
import jax.numpy as jnp
from jax import lax
from jax.experimental import pallas as pl
from jax.experimental.pallas import tpu as pltpu
from jax.experimental.pallas import tpu_sc as plsc

N_NODES = 10000
N_FEAT = 128
N_CLASS = 64
N_EDGES = 320000

NPAD = 10240
NW = 32
EPW = N_EDGES // NW
K = 80
NCHUNK = EPW // K
RPT = NPAD // 16

_MESH = plsc.VectorSubcoreMesh(core_axis_name="c", subcore_axis_name="s")


@functools.partial(
    pl.kernel,
    mesh=_MESH,
    out_type=jax.ShapeDtypeStruct((2, NPAD), jnp.float32),
    scratch_types=[
        pltpu.VMEM((K,), jnp.int32),
        pltpu.VMEM((K,), jnp.float32),
        pltpu.VMEM_SHARED((NPAD,), jnp.float32),
    ],
)
def _sc_degree(dst_hbm, ones_hbm, zeros_hbm, out_hbm, dst_v, ones_v, acc_sh):
    c = lax.axis_index("c")
    s = lax.axis_index("s")
    r0 = s * RPT
    pltpu.sync_copy(zeros_hbm.at[pl.ds(r0, RPT)], acc_sh.at[pl.ds(r0, RPT)])
    pltpu.sync_copy(ones_hbm.at[pl.ds(0, K)], ones_v)
    plsc.subcore_barrier()
    ebase = (s * 2 + c) * EPW

    def body(i, carry):
        base = ebase + i * K
        pltpu.sync_copy(dst_hbm.at[pl.ds(base, K)], dst_v)
        pltpu.sync_copy(ones_v, acc_sh.at[dst_v], add=True)
        return carry

    lax.fori_loop(0, NCHUNK, body, 0)
    plsc.subcore_barrier()
    pltpu.sync_copy(acc_sh.at[pl.ds(r0, RPT)], out_hbm.at[c, pl.ds(r0, RPT)])


NB = 4
LEAD = 2
NGRP = -(-NCHUNK // NB)


@functools.partial(
    pl.kernel,
    mesh=_MESH,
    out_type=jax.ShapeDtypeStruct((2, NPAD, N_FEAT), jnp.float32),
    scratch_types=(
        [pltpu.VMEM((K,), jnp.int32)] * NB
        + [pltpu.VMEM((K,), jnp.int32)] * NB
        + [pltpu.VMEM((K, N_FEAT), jnp.float32)] * NB
        + [pltpu.VMEM_SHARED((NPAD, N_FEAT), jnp.float32)]
        + [pltpu.SemaphoreType.DMA] * (4 * NB)
    ),
)
def _sc_scatter(g_hbm, src_hbm, dst_hbm, zeros_hbm, out_hbm, *rest):
    srcs = rest[0:NB]
    dsts = rest[NB:2 * NB]
    rows = rest[2 * NB:3 * NB]
    acc_sh = rest[3 * NB]
    sems = rest[3 * NB + 1:]
    ss = sems[0:NB]
    sd = sems[NB:2 * NB]
    sg = sems[2 * NB:3 * NB]
    sc_ = sems[3 * NB:4 * NB]
    c = lax.axis_index("c")
    s = lax.axis_index("s")
    r0 = s * RPT
    pltpu.sync_copy(zeros_hbm.at[pl.ds(r0, RPT)], acc_sh.at[pl.ds(r0, RPT)])
    plsc.subcore_barrier()
    ebase = (s * 2 + c) * EPW

    def src_sl(i):
        return src_hbm.at[pl.ds(ebase + i * K, K)]

    def dst_sl(i):
        return dst_hbm.at[pl.ds(ebase + i * K, K)]

    for b in range(LEAD):
        pltpu.sync_copy(src_sl(b), srcs[b])
        pltpu.async_copy(g_hbm.at[srcs[b]], rows[b], sg[b])
        pltpu.async_copy(dst_sl(b), dsts[b], sd[b])
    for b in range(LEAD, NB):
        pltpu.async_copy(src_sl(b), srcs[b], ss[b])

    def group(jg, carry):
        for b in range(NB):
            i = jg * NB + b
            b2 = (b + LEAD) % NB

            @pl.when(i < NCHUNK)
            def _():
                pltpu.make_async_copy(g_hbm.at[srcs[b]], rows[b], sg[b]).wait()

                @pl.when(i + NB < NCHUNK)
                def _():
                    pltpu.async_copy(src_sl(i + NB), srcs[b], ss[b])

                pltpu.make_async_copy(dst_sl(i), dsts[b], sd[b]).wait()
                pltpu.async_copy(rows[b], acc_sh.at[dsts[b]], sc_[b],
                                 add=True)

                @pl.when(i >= LEAD)
                def _():
                    pltpu.make_async_copy(
                        rows[b2], acc_sh.at[dsts[b2]], sc_[b2]).wait()

                @pl.when(i + LEAD < NCHUNK)
                def _():
                    pltpu.make_async_copy(
                        src_sl(i + LEAD), srcs[b2], ss[b2]).wait()
                    pltpu.async_copy(g_hbm.at[srcs[b2]], rows[b2], sg[b2])
                    pltpu.async_copy(dst_sl(i + LEAD), dsts[b2], sd[b2])
        return carry

    lax.fori_loop(0, NGRP, group, 0)
    for i in range(NCHUNK - LEAD, NCHUNK):
        b = i % NB
        pltpu.make_async_copy(rows[b], acc_sh.at[dsts[b]], sc_[b]).wait()
    plsc.subcore_barrier()
    pltpu.sync_copy(acc_sh.at[pl.ds(r0, RPT)], out_hbm.at[c, pl.ds(r0, RPT)])


_R = 1024
_GRID = NPAD // _R


def _m0_body(x_ref, w_ref, dinv_ref, g_ref):
    g_ref[...] = dinv_ref[...] * jnp.dot(
        x_ref[...], w_ref[...], preferred_element_type=jnp.float32)


def _mid_body(p_ref, g_ref, dinv_ref, b_ref, w_ref, h_ref, gn_ref):
    dinv = dinv_ref[...]
    h = jnp.maximum(dinv * (p_ref[0] + p_ref[1] + g_ref[...]) + b_ref[...], 0.0)
    h_ref[...] = h
    gn_ref[...] = dinv * jnp.dot(h, w_ref[...], preferred_element_type=jnp.float32)


def _fin_body(p_ref, g_ref, dinv_ref, b_ref, h1_ref, h2_ref, q_ref,
              wout_ref, bout_ref, out_ref, alpha_ref):
    dinv = dinv_ref[...]
    h3 = jnp.maximum(dinv * (p_ref[0] + p_ref[1] + g_ref[...]) + b_ref[...], 0.0)
    h1 = h1_ref[...]
    h2 = h2_ref[...]
    q = q_ref[...]
    s1 = jnp.sum(h1 * q, axis=1, keepdims=True)
    s2 = jnp.sum(h2 * q, axis=1, keepdims=True)
    s3 = jnp.sum(h3 * q, axis=1, keepdims=True)
    m = jnp.maximum(jnp.maximum(s1, s2), s3)
    e1 = jnp.exp(s1 - m)
    e2 = jnp.exp(s2 - m)
    e3 = jnp.exp(s3 - m)
    den = e1 + e2 + e3
    a1 = e1 / den
    a2 = e2 / den
    a3 = e3 / den
    hsum = a1 * h1 + a2 * h2 + a3 * h3
    out_ref[...] = jnp.dot(
        hsum, wout_ref[...], preferred_element_type=jnp.float32) + bout_ref[...]
    cols = lax.broadcasted_iota(jnp.int32, alpha_ref.shape, 1)
    alpha_ref[...] = jnp.where(
        cols == 0, a1, jnp.where(cols == 1, a2, jnp.where(cols == 2, a3, 0.0)))


def _row_spec(width):
    return pl.BlockSpec((_R, width), lambda i: (i, 0))


def _full_spec(shape):
    nd = len(shape)
    return pl.BlockSpec(shape, lambda i: (0,) * nd)


_P_SPEC = pl.BlockSpec((2, _R, N_FEAT), lambda i: (0, i, 0))

_m0 = pl.pallas_call(
    _m0_body,
    grid=(_GRID,),
    in_specs=[_row_spec(N_FEAT), _full_spec((N_FEAT, N_FEAT)), _row_spec(1)],
    out_specs=_row_spec(N_FEAT),
    out_shape=jax.ShapeDtypeStruct((NPAD, N_FEAT), jnp.float32),
)

_mid = pl.pallas_call(
    _mid_body,
    grid=(_GRID,),
    in_specs=[_P_SPEC, _row_spec(N_FEAT), _row_spec(1),
              _full_spec((1, N_FEAT)), _full_spec((N_FEAT, N_FEAT))],
    out_specs=[_row_spec(N_FEAT), _row_spec(N_FEAT)],
    out_shape=[jax.ShapeDtypeStruct((NPAD, N_FEAT), jnp.float32),
               jax.ShapeDtypeStruct((NPAD, N_FEAT), jnp.float32)],
)

_fin = pl.pallas_call(
    _fin_body,
    grid=(_GRID,),
    in_specs=[_P_SPEC, _row_spec(N_FEAT), _row_spec(1),
              _full_spec((1, N_FEAT)), _row_spec(N_FEAT), _row_spec(N_FEAT),
              _full_spec((1, N_FEAT)), _full_spec((N_FEAT, N_CLASS)),
              _full_spec((1, N_CLASS))],
    out_specs=[_row_spec(N_CLASS), _row_spec(N_FEAT)],
    out_shape=[jax.ShapeDtypeStruct((NPAD, N_CLASS), jnp.float32),
               jax.ShapeDtypeStruct((NPAD, N_FEAT), jnp.float32)],
)


def kernel(x, edge_index, W0, b0, W1, b1, W2, b2, q, Wout, bout):
    src = edge_index[0]
    dst = edge_index[1]
    xp = jnp.pad(x, ((0, NPAD - N_NODES), (0, 0)))
    zerosF = jnp.zeros((NPAD, N_FEAT), jnp.float32)
    zerosN = jnp.zeros((NPAD,), jnp.float32)
    onesK = jnp.ones((K,), jnp.float32)

    degp = _sc_degree(dst, onesK, zerosN)
    dinv = lax.rsqrt(degp[0] + degp[1] + 1.0).reshape(NPAD, 1)

    g0 = _m0(xp, W0, dinv)
    p0 = _sc_scatter(g0, src, dst, zerosF)
    h1, g1 = _mid(p0, g0, dinv, b0.reshape(1, -1), W1)
    p1 = _sc_scatter(g1, src, dst, zerosF)
    h2, g2 = _mid(p1, g1, dinv, b1.reshape(1, -1), W2)
    p2 = _sc_scatter(g2, src, dst, zerosF)
    out_pad, alpha_pad = _fin(p2, g2, dinv, b2.reshape(1, -1), h1, h2,
                              q.reshape(1, -1), Wout, bout.reshape(1, -1))
    return out_pad[:N_NODES], alpha_pad[:N_NODES, :3]


# speedup vs baseline: 25.2795x; 1.0761x over previous
import functools

import jax
import jax.numpy as jnp
from jax import lax
from jax.experimental import pallas as pl
from jax.experimental.pallas import tpu as pltpu
from jax.experimental.pallas import tpu_sc as plsc

N_NODES = 10000
N_FEAT = 128
N_CLASS = 64
N_EDGES = 320000

NPAD = 10240
NW = 32
EPW = N_EDGES // NW
K = 80
NCHUNK = EPW // K
RPT = NPAD // 16

_MESH = plsc.VectorSubcoreMesh(core_axis_name="c", subcore_axis_name="s")


_DNB = 4
_DLEAD = 2


@functools.partial(
    pl.kernel,
    mesh=_MESH,
    out_type=jax.ShapeDtypeStruct((2, NPAD), jnp.float32),
    scratch_types=(
        [pltpu.VMEM((K,), jnp.float32)]
        + [pltpu.VMEM((K,), jnp.int32)] * _DNB
        + [pltpu.VMEM_SHARED((NPAD,), jnp.float32)]
        + [pltpu.SemaphoreType.DMA] * (2 * _DNB)
    ),
)
def _sc_degree(dst_hbm, ones_hbm, zeros_hbm, out_hbm, ones_v, *rest):
    dsts = rest[0:_DNB]
    acc_sh = rest[_DNB]
    sems = rest[_DNB + 1:]
    sd = sems[0:_DNB]
    sc_ = sems[_DNB:2 * _DNB]
    c = lax.axis_index("c")
    s = lax.axis_index("s")
    r0 = s * RPT
    pltpu.sync_copy(zeros_hbm.at[pl.ds(r0, RPT)], acc_sh.at[pl.ds(r0, RPT)])
    pltpu.sync_copy(ones_hbm.at[pl.ds(0, K)], ones_v)
    plsc.subcore_barrier()
    ebase = (s * 2 + c) * EPW

    def dst_sl(i):
        return dst_hbm.at[pl.ds(ebase + i * K, K)]

    for b in range(_DLEAD):
        pltpu.async_copy(dst_sl(b), dsts[b], sd[b])

    def body(jg, carry):
        for b in range(_DNB):
            i = jg * _DNB + b
            b2 = (b + _DLEAD) % _DNB

            @pl.when(i < NCHUNK)
            def _():
                pltpu.make_async_copy(dst_sl(i), dsts[b], sd[b]).wait()
                pltpu.async_copy(ones_v, acc_sh.at[dsts[b]], sc_[b], add=True)

                @pl.when(i >= _DLEAD)
                def _():
                    pltpu.make_async_copy(
                        ones_v, acc_sh.at[dsts[b2]], sc_[b2]).wait()

                @pl.when(i + _DLEAD < NCHUNK)
                def _():
                    pltpu.async_copy(dst_sl(i + _DLEAD), dsts[b2], sd[b2])
        return carry

    lax.fori_loop(0, -(-NCHUNK // _DNB), body, 0)
    for i in range(NCHUNK - _DLEAD, NCHUNK):
        b = i % _DNB
        pltpu.make_async_copy(ones_v, acc_sh.at[dsts[b]], sc_[b]).wait()
    plsc.subcore_barrier()
    pltpu.sync_copy(acc_sh.at[pl.ds(r0, RPT)], out_hbm.at[c, pl.ds(r0, RPT)])


NB = 4
LEAD = 2
NGRP = -(-NCHUNK // NB)


@functools.partial(
    pl.kernel,
    mesh=_MESH,
    out_type=jax.ShapeDtypeStruct((2, NPAD, N_FEAT), jnp.float32),
    scratch_types=(
        [pltpu.VMEM((K,), jnp.int32)] * NB
        + [pltpu.VMEM((K,), jnp.int32)] * NB
        + [pltpu.VMEM((K, N_FEAT), jnp.float32)] * NB
        + [pltpu.VMEM_SHARED((NPAD, N_FEAT), jnp.float32)]
        + [pltpu.SemaphoreType.DMA] * (4 * NB)
    ),
)
def _sc_scatter(g_hbm, src_hbm, dst_hbm, zeros_hbm, out_hbm, *rest):
    srcs = rest[0:NB]
    dsts = rest[NB:2 * NB]
    rows = rest[2 * NB:3 * NB]
    acc_sh = rest[3 * NB]
    sems = rest[3 * NB + 1:]
    ss = sems[0:NB]
    sd = sems[NB:2 * NB]
    sg = sems[2 * NB:3 * NB]
    sc_ = sems[3 * NB:4 * NB]
    c = lax.axis_index("c")
    s = lax.axis_index("s")
    r0 = s * RPT
    pltpu.sync_copy(zeros_hbm.at[pl.ds(r0, RPT)], acc_sh.at[pl.ds(r0, RPT)])
    plsc.subcore_barrier()
    ebase = (s * 2 + c) * EPW

    def src_sl(i):
        return src_hbm.at[pl.ds(ebase + i * K, K)]

    def dst_sl(i):
        return dst_hbm.at[pl.ds(ebase + i * K, K)]

    for b in range(LEAD):
        pltpu.sync_copy(src_sl(b), srcs[b])
        pltpu.async_copy(g_hbm.at[srcs[b]], rows[b], sg[b])
        pltpu.async_copy(dst_sl(b), dsts[b], sd[b])
    for b in range(LEAD, NB):
        pltpu.async_copy(src_sl(b), srcs[b], ss[b])

    def group(jg, carry):
        for b in range(NB):
            i = jg * NB + b
            b2 = (b + LEAD) % NB

            @pl.when(i < NCHUNK)
            def _():
                pltpu.make_async_copy(g_hbm.at[srcs[b]], rows[b], sg[b]).wait()

                @pl.when(i + NB < NCHUNK)
                def _():
                    pltpu.async_copy(src_sl(i + NB), srcs[b], ss[b])

                pltpu.make_async_copy(dst_sl(i), dsts[b], sd[b]).wait()
                pltpu.async_copy(rows[b], acc_sh.at[dsts[b]], sc_[b],
                                 add=True)

                @pl.when(i >= LEAD)
                def _():
                    pltpu.make_async_copy(
                        rows[b2], acc_sh.at[dsts[b2]], sc_[b2]).wait()

                @pl.when(i + LEAD < NCHUNK)
                def _():
                    pltpu.make_async_copy(
                        src_sl(i + LEAD), srcs[b2], ss[b2]).wait()
                    pltpu.async_copy(g_hbm.at[srcs[b2]], rows[b2], sg[b2])
                    pltpu.async_copy(dst_sl(i + LEAD), dsts[b2], sd[b2])
        return carry

    lax.fori_loop(0, NGRP, group, 0)
    for i in range(NCHUNK - LEAD, NCHUNK):
        b = i % NB
        pltpu.make_async_copy(rows[b], acc_sh.at[dsts[b]], sc_[b]).wait()
    plsc.subcore_barrier()
    pltpu.sync_copy(acc_sh.at[pl.ds(r0, RPT)], out_hbm.at[c, pl.ds(r0, RPT)])


_R = 1024
_GRID = NPAD // _R


def _m0_body(x_ref, w_ref, dinv_ref, g_ref):
    g_ref[...] = dinv_ref[...] * jnp.dot(
        x_ref[...], w_ref[...], preferred_element_type=jnp.float32)


def _mid_body(p_ref, g_ref, dinv_ref, b_ref, w_ref, h_ref, gn_ref):
    dinv = dinv_ref[...]
    h = jnp.maximum(dinv * (p_ref[0] + p_ref[1] + g_ref[...]) + b_ref[...], 0.0)
    h_ref[...] = h
    gn_ref[...] = dinv * jnp.dot(h, w_ref[...], preferred_element_type=jnp.float32)


def _fin_body(p_ref, g_ref, dinv_ref, b_ref, h1_ref, h2_ref, q_ref,
              wout_ref, bout_ref, out_ref, alpha_ref):
    dinv = dinv_ref[...]
    h3 = jnp.maximum(dinv * (p_ref[0] + p_ref[1] + g_ref[...]) + b_ref[...], 0.0)
    h1 = h1_ref[...]
    h2 = h2_ref[...]
    q = q_ref[...]
    s1 = jnp.sum(h1 * q, axis=1, keepdims=True)
    s2 = jnp.sum(h2 * q, axis=1, keepdims=True)
    s3 = jnp.sum(h3 * q, axis=1, keepdims=True)
    m = jnp.maximum(jnp.maximum(s1, s2), s3)
    e1 = jnp.exp(s1 - m)
    e2 = jnp.exp(s2 - m)
    e3 = jnp.exp(s3 - m)
    den = e1 + e2 + e3
    a1 = e1 / den
    a2 = e2 / den
    a3 = e3 / den
    hsum = a1 * h1 + a2 * h2 + a3 * h3
    out_ref[...] = jnp.dot(
        hsum, wout_ref[...], preferred_element_type=jnp.float32) + bout_ref[...]
    cols = lax.broadcasted_iota(jnp.int32, alpha_ref.shape, 1)
    alpha_ref[...] = jnp.where(
        cols == 0, a1, jnp.where(cols == 1, a2, jnp.where(cols == 2, a3, 0.0)))


def _row_spec(width):
    return pl.BlockSpec((_R, width), lambda i: (i, 0))


def _full_spec(shape):
    nd = len(shape)
    return pl.BlockSpec(shape, lambda i: (0,) * nd)


_P_SPEC = pl.BlockSpec((2, _R, N_FEAT), lambda i: (0, i, 0))

_m0 = pl.pallas_call(
    _m0_body,
    grid=(_GRID,),
    in_specs=[_row_spec(N_FEAT), _full_spec((N_FEAT, N_FEAT)), _row_spec(1)],
    out_specs=_row_spec(N_FEAT),
    out_shape=jax.ShapeDtypeStruct((NPAD, N_FEAT), jnp.float32),
)

_mid = pl.pallas_call(
    _mid_body,
    grid=(_GRID,),
    in_specs=[_P_SPEC, _row_spec(N_FEAT), _row_spec(1),
              _full_spec((1, N_FEAT)), _full_spec((N_FEAT, N_FEAT))],
    out_specs=[_row_spec(N_FEAT), _row_spec(N_FEAT)],
    out_shape=[jax.ShapeDtypeStruct((NPAD, N_FEAT), jnp.float32),
               jax.ShapeDtypeStruct((NPAD, N_FEAT), jnp.float32)],
)

_fin = pl.pallas_call(
    _fin_body,
    grid=(_GRID,),
    in_specs=[_P_SPEC, _row_spec(N_FEAT), _row_spec(1),
              _full_spec((1, N_FEAT)), _row_spec(N_FEAT), _row_spec(N_FEAT),
              _full_spec((1, N_FEAT)), _full_spec((N_FEAT, N_CLASS)),
              _full_spec((1, N_CLASS))],
    out_specs=[_row_spec(N_CLASS), _row_spec(N_FEAT)],
    out_shape=[jax.ShapeDtypeStruct((NPAD, N_CLASS), jnp.float32),
               jax.ShapeDtypeStruct((NPAD, N_FEAT), jnp.float32)],
)


def kernel(x, edge_index, W0, b0, W1, b1, W2, b2, q, Wout, bout):
    src = edge_index[0]
    dst = edge_index[1]
    xp = jnp.pad(x, ((0, NPAD - N_NODES), (0, 0)))
    zerosF = jnp.zeros((NPAD, N_FEAT), jnp.float32)
    zerosN = jnp.zeros((NPAD,), jnp.float32)
    onesK = jnp.ones((K,), jnp.float32)

    degp = _sc_degree(dst, onesK, zerosN)
    dinv = lax.rsqrt(degp[0] + degp[1] + 1.0).reshape(NPAD, 1)

    g0 = _m0(xp, W0, dinv)
    p0 = _sc_scatter(g0, src, dst, zerosF)
    h1, g1 = _mid(p0, g0, dinv, b0.reshape(1, -1), W1)
    p1 = _sc_scatter(g1, src, dst, zerosF)
    h2, g2 = _mid(p1, g1, dinv, b1.reshape(1, -1), W2)
    p2 = _sc_scatter(g2, src, dst, zerosF)
    out_pad, alpha_pad = _fin(p2, g2, dinv, b2.reshape(1, -1), h1, h2,
                              q.reshape(1, -1), Wout, bout.reshape(1, -1))
    return out_pad[:N_NODES], alpha_pad[:N_NODES, :3]
